# baseline (speedup 1.0000x reference)
import functools

import jax
import jax.numpy as jnp
from jax import lax
from jax.experimental import pallas as pl
from jax.experimental.pallas import tpu as pltpu
from jax.experimental.pallas import tpu_sc as plsc

V = 1000
VC = 1024
D = 128
EPS = 0.2
T_INV = 2.0
BLK = 512

NC = 2
NS = 16
NW = NC * NS
CHUNK = 16
LANES = 16


def _s_table_body(emb_ref, s_ref):
    emb = emb_ref[...]
    ss = jnp.sum(emb * emb, axis=1, keepdims=True)
    nrm = jnp.maximum(jnp.sqrt(ss), 1e-12)
    emb_n = emb / nrm
    sim = jax.lax.dot_general(
        emb_n, emb_n, (((1,), (1,)), ((), ())),
        preferred_element_type=jnp.float32)
    rows = jax.lax.broadcasted_iota(jnp.int32, (V, V), 0)
    cols = jax.lax.broadcasted_iota(jnp.int32, (V, V), 1)
    diag = rows == cols
    masked = jnp.where(diag, -jnp.inf, sim * T_INV)
    m = jnp.max(masked, axis=1, keepdims=True)
    e = jnp.exp(masked - m)
    denom = jnp.sum(e, axis=1, keepdims=True)
    s = e * (EPS / denom)
    s = jnp.where(diag, 1.0 - EPS, s)
    s_ref[...] = jnp.pad(s, ((0, 0), (0, VC - V)))


def _make_dot_gather(batch):
    rows_per_w = batch // NW
    nchunk = rows_per_w // CHUNK
    mesh = plsc.VectorSubcoreMesh(core_axis_name="c", subcore_axis_name="s")

    @functools.partial(
        pl.kernel,
        mesh=mesh,
        out_type=jax.ShapeDtypeStruct((NW, LANES), jnp.float32),
        scratch_types=[
            pltpu.VMEM((CHUNK,), jnp.int32),
            pltpu.VMEM((CHUNK,), jnp.int32),
            pltpu.VMEM((CHUNK, V), jnp.float32),
            pltpu.VMEM((CHUNK, V), jnp.float32),
            pltpu.VMEM((CHUNK, VC), jnp.float32),
            pltpu.VMEM((CHUNK, VC), jnp.float32),
            pltpu.VMEM((LANES,), jnp.float32),
            pltpu.SemaphoreType.DMA,
            pltpu.SemaphoreType.DMA,
        ],
    )
    def dot_gather(logits_hbm, labels_hbm, s_hbm, out_hbm,
                   idx_a, idx_b, x_a, x_b, sr_a, sr_b, acc_v, sem_a, sem_b):
        c = lax.axis_index("c")
        s = lax.axis_index("s")
        wid = c * NS + s

        iota = jax.lax.broadcasted_iota(jnp.int32, (LANES,), 0)
        tail_mask = jnp.where(iota >= 8,
                              jnp.ones((LANES,), jnp.float32),
                              jnp.zeros((LANES,), jnp.float32))

        def mac(x_v, srows_v, acc):
            def rbody(r, acc_in):
                a = acc_in
                for j in range(V // LANES):
                    x = x_v[r, pl.ds(j * LANES, LANES)]
                    sr = srows_v[r, pl.ds(j * LANES, LANES)]
                    a = a + x * sr
                xt = x_v[r, pl.ds(V - LANES, LANES)] * tail_mask
                st = srows_v[r, pl.ds(V - LANES, LANES)]
                return a + xt * st

            return lax.fori_loop(0, CHUNK, rbody, acc)

        def body(g, acc):
            base = wid * rows_per_w + 2 * g * CHUNK
            pltpu.sync_copy(labels_hbm.at[pl.ds(base, CHUNK)], idx_a)
            pltpu.sync_copy(labels_hbm.at[pl.ds(base + CHUNK, CHUNK)], idx_b)
            stg_a = pltpu.async_copy(
                logits_hbm.at[pl.ds(base, CHUNK)], x_a, sem_a)
            gat_a = pltpu.async_copy(s_hbm.at[idx_a], sr_a, sem_a)
            stg_b = pltpu.async_copy(
                logits_hbm.at[pl.ds(base + CHUNK, CHUNK)], x_b, sem_b)
            gat_b = pltpu.async_copy(s_hbm.at[idx_b], sr_b, sem_b)
            stg_a.wait()
            gat_a.wait()
            acc = mac(x_a, sr_a, acc)
            stg_b.wait()
            gat_b.wait()
            return mac(x_b, sr_b, acc)

        acc = lax.fori_loop(0, nchunk // 2, body,
                            jnp.zeros((LANES,), jnp.float32))
        acc_v[...] = acc
        pltpu.sync_copy(acc_v, out_hbm.at[wid])

    return dot_gather


def _lse_body(logits_ref, out_ref, acc_scr):
    i = pl.program_id(0)

    @pl.when(i == 0)
    def _init():
        acc_scr[0] = 0.0

    x = logits_ref[...]
    m = jnp.max(x, axis=1, keepdims=True)
    e = jnp.exp(x - m)
    sm = jnp.sum(e, axis=1)
    lse = jnp.log(sm) + m[:, 0]
    acc_scr[0] += jnp.sum(lse)

    @pl.when(i == pl.num_programs(0) - 1)
    def _fin():
        out_ref[0] = acc_scr[0]


def kernel(logits, labels, word_emb_tab):
    logits = logits.astype(jnp.float32)
    labels = labels.astype(jnp.int32)
    batch = logits.shape[0]
    nblk = batch // BLK

    s_tab = pl.pallas_call(
        _s_table_body,
        out_shape=jax.ShapeDtypeStruct((V, VC), jnp.float32),
    )(word_emb_tab.astype(jnp.float32))

    dots = _make_dot_gather(batch)(logits, labels, s_tab)

    lse_total = pl.pallas_call(
        _lse_body,
        grid=(nblk,),
        in_specs=[
            pl.BlockSpec((BLK, V), lambda i: (i, 0)),
        ],
        out_specs=pl.BlockSpec(memory_space=pltpu.SMEM),
        out_shape=jax.ShapeDtypeStruct((1,), jnp.float32),
        scratch_shapes=[
            pltpu.SMEM((1,), jnp.float32),
        ],
    )(logits)

    return ((lse_total[0] - jnp.sum(dots)) / batch).astype(jnp.float32)


# speedup vs baseline: 1.0305x; 1.0305x over previous
import functools

import jax
import jax.numpy as jnp
from jax import lax
from jax.experimental import pallas as pl
from jax.experimental.pallas import tpu as pltpu
from jax.experimental.pallas import tpu_sc as plsc

V = 1000
VC = 1024
D = 128
EPS = 0.2
T_INV = 2.0
BLK = 512

NC = 2
NS = 16
NW = NC * NS
CHUNK = 32
LANES = 16


def _s_table_body(emb_ref, s_ref):
    emb = emb_ref[...]
    ss = jnp.sum(emb * emb, axis=1, keepdims=True)
    nrm = jnp.maximum(jnp.sqrt(ss), 1e-12)
    emb_n = emb / nrm
    sim = jax.lax.dot_general(
        emb_n, emb_n, (((1,), (1,)), ((), ())),
        preferred_element_type=jnp.float32)
    rows = jax.lax.broadcasted_iota(jnp.int32, (V, V), 0)
    cols = jax.lax.broadcasted_iota(jnp.int32, (V, V), 1)
    diag = rows == cols
    masked = jnp.where(diag, -jnp.inf, sim * T_INV)
    m = jnp.max(masked, axis=1, keepdims=True)
    e = jnp.exp(masked - m)
    denom = jnp.sum(e, axis=1, keepdims=True)
    s = e * (EPS / denom)
    s = jnp.where(diag, 1.0 - EPS, s)
    s_ref[...] = jnp.pad(s, ((0, 0), (0, VC - V)))


def _make_dot_gather(batch):
    rows_per_w = batch // NW
    nchunk = rows_per_w // CHUNK
    mesh = plsc.VectorSubcoreMesh(core_axis_name="c", subcore_axis_name="s")

    @functools.partial(
        pl.kernel,
        mesh=mesh,
        out_type=jax.ShapeDtypeStruct((NW, LANES), jnp.float32),
        scratch_types=[
            pltpu.VMEM((CHUNK,), jnp.int32),
            pltpu.VMEM((CHUNK, V), jnp.float32),
            pltpu.VMEM((CHUNK, VC), jnp.float32),
            pltpu.VMEM((LANES,), jnp.float32),
            pltpu.SemaphoreType.DMA,
        ],
    )
    def dot_gather(logits_hbm, labels_hbm, s_hbm, out_hbm,
                   idx_v, x_v, srows_v, acc_v, sem):
        c = lax.axis_index("c")
        s = lax.axis_index("s")
        wid = c * NS + s

        iota = jax.lax.broadcasted_iota(jnp.int32, (LANES,), 0)
        tail_mask = jnp.where(iota >= 8,
                              jnp.ones((LANES,), jnp.float32),
                              jnp.zeros((LANES,), jnp.float32))

        def body(k, acc):
            base = wid * rows_per_w + k * CHUNK
            pltpu.sync_copy(labels_hbm.at[pl.ds(base, CHUNK)], idx_v)
            stg = pltpu.async_copy(
                logits_hbm.at[pl.ds(base, CHUNK)], x_v, sem)
            gat = pltpu.async_copy(s_hbm.at[idx_v], srows_v, sem)
            stg.wait()
            gat.wait()

            def rbody(r, acc_in):
                a = acc_in
                for j in range(V // LANES):
                    x = x_v[r, pl.ds(j * LANES, LANES)]
                    sr = srows_v[r, pl.ds(j * LANES, LANES)]
                    a = a + x * sr
                xt = x_v[r, pl.ds(V - LANES, LANES)] * tail_mask
                st = srows_v[r, pl.ds(V - LANES, LANES)]
                return a + xt * st

            return lax.fori_loop(0, CHUNK, rbody, acc)

        acc = lax.fori_loop(0, nchunk, body, jnp.zeros((LANES,), jnp.float32))
        acc_v[...] = acc
        pltpu.sync_copy(acc_v, out_hbm.at[wid])

    return dot_gather


def _lse_body(logits_ref, out_ref, acc_scr):
    i = pl.program_id(0)

    @pl.when(i == 0)
    def _init():
        acc_scr[0] = 0.0

    x = logits_ref[...]
    m = jnp.max(x, axis=1, keepdims=True)
    e = jnp.exp(x - m)
    sm = jnp.sum(e, axis=1)
    lse = jnp.log(sm) + m[:, 0]
    acc_scr[0] += jnp.sum(lse)

    @pl.when(i == pl.num_programs(0) - 1)
    def _fin():
        out_ref[0] = acc_scr[0]


def kernel(logits, labels, word_emb_tab):
    logits = logits.astype(jnp.float32)
    labels = labels.astype(jnp.int32)
    batch = logits.shape[0]
    nblk = batch // BLK

    s_tab = pl.pallas_call(
        _s_table_body,
        out_shape=jax.ShapeDtypeStruct((V, VC), jnp.float32),
    )(word_emb_tab.astype(jnp.float32))

    dots = _make_dot_gather(batch)(logits, labels, s_tab)

    lse_total = pl.pallas_call(
        _lse_body,
        grid=(nblk,),
        in_specs=[
            pl.BlockSpec((BLK, V), lambda i: (i, 0)),
        ],
        out_specs=pl.BlockSpec(memory_space=pltpu.SMEM),
        out_shape=jax.ShapeDtypeStruct((1,), jnp.float32),
        scratch_shapes=[
            pltpu.SMEM((1,), jnp.float32),
        ],
    )(logits)

    return ((lse_total[0] - jnp.sum(dots)) / batch).astype(jnp.float32)


# speedup vs baseline: 1.2257x; 1.1894x over previous
import functools

import jax
import jax.numpy as jnp
from jax import lax
from jax.experimental import pallas as pl
from jax.experimental.pallas import tpu as pltpu
from jax.experimental.pallas import tpu_sc as plsc

V = 1000
VC = 1024
D = 128
EPS = 0.2
T_INV = 2.0
BLK = 512

NC = 2
NS = 16
NW = NC * NS
CHUNK = 32
TC_BLOCKS = 10
LANES = 16


def _s_table_body(emb_ref, s_ref, sbf_ref):
    emb = emb_ref[...]
    ss = jnp.sum(emb * emb, axis=1, keepdims=True)
    nrm = jnp.maximum(jnp.sqrt(ss), 1e-12)
    emb_n = emb / nrm
    sim = jax.lax.dot_general(
        emb_n, emb_n, (((1,), (1,)), ((), ())),
        preferred_element_type=jnp.float32)
    rows = jax.lax.broadcasted_iota(jnp.int32, (V, V), 0)
    cols = jax.lax.broadcasted_iota(jnp.int32, (V, V), 1)
    diag = rows == cols
    masked = jnp.where(diag, -jnp.inf, sim * T_INV)
    m = jnp.max(masked, axis=1, keepdims=True)
    e = jnp.exp(masked - m)
    denom = jnp.sum(e, axis=1, keepdims=True)
    s = e * (EPS / denom)
    s = jnp.where(diag, 1.0 - EPS, s)
    s_ref[...] = jnp.pad(s, ((0, 0), (0, VC - V)))
    sbf_ref[...] = s.astype(jnp.bfloat16)


def _make_dot_gather(batch):
    row0 = TC_BLOCKS * BLK
    rows_per_w = (batch - row0) // NW
    nchunk = rows_per_w // CHUNK
    mesh = plsc.VectorSubcoreMesh(core_axis_name="c", subcore_axis_name="s")

    @functools.partial(
        pl.kernel,
        mesh=mesh,
        out_type=jax.ShapeDtypeStruct((NW, LANES), jnp.float32),
        scratch_types=[
            pltpu.VMEM((CHUNK,), jnp.int32),
            pltpu.VMEM((CHUNK, V), jnp.float32),
            pltpu.VMEM((CHUNK, VC), jnp.float32),
            pltpu.VMEM((LANES,), jnp.float32),
            pltpu.SemaphoreType.DMA,
        ],
    )
    def dot_gather(logits_hbm, labels_hbm, s_hbm, out_hbm,
                   idx_v, x_v, srows_v, acc_v, sem):
        c = lax.axis_index("c")
        s = lax.axis_index("s")
        wid = c * NS + s

        iota = jax.lax.broadcasted_iota(jnp.int32, (LANES,), 0)
        tail_mask = jnp.where(iota >= 8,
                              jnp.ones((LANES,), jnp.float32),
                              jnp.zeros((LANES,), jnp.float32))

        def body(k, acc):
            base = row0 + wid * rows_per_w + k * CHUNK
            pltpu.sync_copy(labels_hbm.at[pl.ds(base, CHUNK)], idx_v)
            stg = pltpu.async_copy(
                logits_hbm.at[pl.ds(base, CHUNK)], x_v, sem)
            gat = pltpu.async_copy(s_hbm.at[idx_v], srows_v, sem)
            stg.wait()
            gat.wait()

            def rbody(r, acc_in):
                a = acc_in
                for j in range(V // LANES):
                    x = x_v[r, pl.ds(j * LANES, LANES)]
                    sr = srows_v[r, pl.ds(j * LANES, LANES)]
                    a = a + x * sr
                xt = x_v[r, pl.ds(V - LANES, LANES)] * tail_mask
                st = srows_v[r, pl.ds(V - LANES, LANES)]
                return a + xt * st

            return lax.fori_loop(0, CHUNK, rbody, acc)

        acc = lax.fori_loop(0, nchunk, body, jnp.zeros((LANES,), jnp.float32))
        acc_v[...] = acc
        pltpu.sync_copy(acc_v, out_hbm.at[wid])

    return dot_gather


def _lse_body(logits_ref, labels_ref, sbf_ref, out_ref, acc_scr):
    i = pl.program_id(0)

    @pl.when(i == 0)
    def _init():
        acc_scr[0] = 0.0

    x = logits_ref[...]
    m = jnp.max(x, axis=1, keepdims=True)
    e = jnp.exp(x - m)
    sm = jnp.sum(e, axis=1)
    lse = jnp.log(sm) + m[:, 0]
    acc_scr[0] += jnp.sum(lse)

    @pl.when(i < TC_BLOCKS)
    def _dot():
        lab = labels_ref[0, 0, :]
        oh = (jax.lax.broadcasted_iota(jnp.int32, (BLK, V), 1)
              == lab[:, None]).astype(jnp.bfloat16)
        sg = jax.lax.dot_general(
            oh, sbf_ref[...], (((1,), (0,)), ((), ())),
            preferred_element_type=jnp.float32)
        acc_scr[0] += -jnp.sum(sg * x)

    @pl.when(i == pl.num_programs(0) - 1)
    def _fin():
        out_ref[0] = acc_scr[0]


def kernel(logits, labels, word_emb_tab):
    logits = logits.astype(jnp.float32)
    labels = labels.astype(jnp.int32)
    batch = logits.shape[0]
    nblk = batch // BLK

    s_tab, s_bf = pl.pallas_call(
        _s_table_body,
        out_shape=(jax.ShapeDtypeStruct((V, VC), jnp.float32),
                   jax.ShapeDtypeStruct((V, V), jnp.bfloat16)),
    )(word_emb_tab.astype(jnp.float32))

    dots = _make_dot_gather(batch)(logits, labels, s_tab)

    labels3 = labels.reshape(nblk, 1, BLK)
    lse_total = pl.pallas_call(
        _lse_body,
        grid=(nblk,),
        in_specs=[
            pl.BlockSpec((BLK, V), lambda i: (i, 0)),
            pl.BlockSpec((1, 1, BLK), lambda i: (i, 0, 0)),
            pl.BlockSpec((V, V), lambda i: (0, 0)),
        ],
        out_specs=pl.BlockSpec(memory_space=pltpu.SMEM),
        out_shape=jax.ShapeDtypeStruct((1,), jnp.float32),
        scratch_shapes=[
            pltpu.SMEM((1,), jnp.float32),
        ],
    )(logits, labels3, s_bf)

    return ((lse_total[0] - jnp.sum(dots)) / batch).astype(jnp.float32)


# speedup vs baseline: 1.2810x; 1.0451x over previous
import functools

import jax
import jax.numpy as jnp
from jax import lax
from jax.experimental import pallas as pl
from jax.experimental.pallas import tpu as pltpu
from jax.experimental.pallas import tpu_sc as plsc

V = 1000
VC = 1024
D = 128
EPS = 0.2
T_INV = 2.0
BLK = 512

NC = 2
NS = 16
NW = NC * NS
CHUNK = 32
TC_BLOCKS = 12
LANES = 16


def _s_table_body(emb_ref, s_ref, sbf_ref):
    emb = emb_ref[...]
    ss = jnp.sum(emb * emb, axis=1, keepdims=True)
    nrm = jnp.maximum(jnp.sqrt(ss), 1e-12)
    emb_n = emb / nrm
    sim = jax.lax.dot_general(
        emb_n, emb_n, (((1,), (1,)), ((), ())),
        preferred_element_type=jnp.float32)
    rows = jax.lax.broadcasted_iota(jnp.int32, (V, V), 0)
    cols = jax.lax.broadcasted_iota(jnp.int32, (V, V), 1)
    diag = rows == cols
    masked = jnp.where(diag, -jnp.inf, sim * T_INV)
    m = jnp.max(masked, axis=1, keepdims=True)
    e = jnp.exp(masked - m)
    denom = jnp.sum(e, axis=1, keepdims=True)
    s = e * (EPS / denom)
    s = jnp.where(diag, 1.0 - EPS, s)
    s_ref[...] = jnp.pad(s, ((0, 0), (0, VC - V)))
    sbf_ref[...] = s.astype(jnp.bfloat16)


def _make_dot_gather(batch):
    row0 = TC_BLOCKS * BLK
    rows_per_w = (batch - row0) // NW
    nchunk = rows_per_w // CHUNK
    mesh = plsc.VectorSubcoreMesh(core_axis_name="c", subcore_axis_name="s")

    @functools.partial(
        pl.kernel,
        mesh=mesh,
        out_type=jax.ShapeDtypeStruct((NW, LANES), jnp.float32),
        scratch_types=[
            pltpu.VMEM((CHUNK,), jnp.int32),
            pltpu.VMEM((CHUNK, V), jnp.float32),
            pltpu.VMEM((CHUNK, VC), jnp.float32),
            pltpu.VMEM((LANES,), jnp.float32),
            pltpu.SemaphoreType.DMA,
        ],
    )
    def dot_gather(logits_hbm, labels_hbm, s_hbm, out_hbm,
                   idx_v, x_v, srows_v, acc_v, sem):
        c = lax.axis_index("c")
        s = lax.axis_index("s")
        wid = c * NS + s

        iota = jax.lax.broadcasted_iota(jnp.int32, (LANES,), 0)
        tail_mask = jnp.where(iota >= 8,
                              jnp.ones((LANES,), jnp.float32),
                              jnp.zeros((LANES,), jnp.float32))

        def body(k, acc):
            base = row0 + wid * rows_per_w + k * CHUNK
            pltpu.sync_copy(labels_hbm.at[pl.ds(base, CHUNK)], idx_v)
            stg = pltpu.async_copy(
                logits_hbm.at[pl.ds(base, CHUNK)], x_v, sem)
            gat = pltpu.async_copy(s_hbm.at[idx_v], srows_v, sem)
            stg.wait()
            gat.wait()

            def rbody(r, acc_in):
                a = acc_in
                for j in range(V // LANES):
                    x = x_v[r, pl.ds(j * LANES, LANES)]
                    sr = srows_v[r, pl.ds(j * LANES, LANES)]
                    a = a + x * sr
                xt = x_v[r, pl.ds(V - LANES, LANES)] * tail_mask
                st = srows_v[r, pl.ds(V - LANES, LANES)]
                return a + xt * st

            return lax.fori_loop(0, CHUNK, rbody, acc)

        acc = lax.fori_loop(0, nchunk, body, jnp.zeros((LANES,), jnp.float32))
        acc_v[...] = acc
        pltpu.sync_copy(acc_v, out_hbm.at[wid])

    return dot_gather


def _lse_body(logits_ref, labels_ref, sbf_ref, out_ref, acc_scr):
    i = pl.program_id(0)

    @pl.when(i == 0)
    def _init():
        acc_scr[0] = 0.0

    x = logits_ref[...]
    m = jnp.max(x, axis=1, keepdims=True)
    e = jnp.exp(x - m)
    sm = jnp.sum(e, axis=1)
    lse = jnp.log(sm) + m[:, 0]
    acc_scr[0] += jnp.sum(lse)

    @pl.when(i < TC_BLOCKS)
    def _dot():
        lab = labels_ref[0, 0, :]
        oh = (jax.lax.broadcasted_iota(jnp.int32, (BLK, V), 1)
              == lab[:, None]).astype(jnp.bfloat16)
        sg = jax.lax.dot_general(
            oh, sbf_ref[...], (((1,), (0,)), ((), ())),
            preferred_element_type=jnp.float32)
        acc_scr[0] += -jnp.sum(sg * x)

    @pl.when(i == pl.num_programs(0) - 1)
    def _fin():
        out_ref[0] = acc_scr[0]


def kernel(logits, labels, word_emb_tab):
    logits = logits.astype(jnp.float32)
    labels = labels.astype(jnp.int32)
    batch = logits.shape[0]
    nblk = batch // BLK

    s_tab, s_bf = pl.pallas_call(
        _s_table_body,
        out_shape=(jax.ShapeDtypeStruct((V, VC), jnp.float32),
                   jax.ShapeDtypeStruct((V, V), jnp.bfloat16)),
    )(word_emb_tab.astype(jnp.float32))

    dots = _make_dot_gather(batch)(logits, labels, s_tab)

    labels3 = labels.reshape(nblk, 1, BLK)
    lse_total = pl.pallas_call(
        _lse_body,
        grid=(nblk,),
        in_specs=[
            pl.BlockSpec((BLK, V), lambda i: (i, 0)),
            pl.BlockSpec((1, 1, BLK), lambda i: (i, 0, 0)),
            pl.BlockSpec((V, V), lambda i: (0, 0)),
        ],
        out_specs=pl.BlockSpec(memory_space=pltpu.SMEM),
        out_shape=jax.ShapeDtypeStruct((1,), jnp.float32),
        scratch_shapes=[
            pltpu.SMEM((1,), jnp.float32),
        ],
    )(logits, labels3, s_bf)

    return ((lse_total[0] - jnp.sum(dots)) / batch).astype(jnp.float32)


# speedup vs baseline: 1.3204x; 1.0308x over previous
import functools

import jax
import jax.numpy as jnp
from jax import lax
from jax.experimental import pallas as pl
from jax.experimental.pallas import tpu as pltpu
from jax.experimental.pallas import tpu_sc as plsc

V = 1000
VC = 1024
D = 128
EPS = 0.2
T_INV = 2.0
BLK = 512

NC = 2
NS = 16
NW = NC * NS
CHUNK = 32
TC_BLOCKS = 14
LANES = 16


def _s_table_body(emb_ref, s_ref, sbf_ref):
    emb = emb_ref[...]
    ss = jnp.sum(emb * emb, axis=1, keepdims=True)
    nrm = jnp.maximum(jnp.sqrt(ss), 1e-12)
    emb_n = emb / nrm
    sim = jax.lax.dot_general(
        emb_n, emb_n, (((1,), (1,)), ((), ())),
        preferred_element_type=jnp.float32)
    rows = jax.lax.broadcasted_iota(jnp.int32, (V, V), 0)
    cols = jax.lax.broadcasted_iota(jnp.int32, (V, V), 1)
    diag = rows == cols
    masked = jnp.where(diag, -jnp.inf, sim * T_INV)
    m = jnp.max(masked, axis=1, keepdims=True)
    e = jnp.exp(masked - m)
    denom = jnp.sum(e, axis=1, keepdims=True)
    s = e * (EPS / denom)
    s = jnp.where(diag, 1.0 - EPS, s)
    s_ref[...] = jnp.pad(s, ((0, 0), (0, VC - V)))
    sbf_ref[...] = s.astype(jnp.bfloat16)


def _make_dot_gather(batch):
    row0 = TC_BLOCKS * BLK
    rows_per_w = (batch - row0) // NW
    nchunk = rows_per_w // CHUNK
    mesh = plsc.VectorSubcoreMesh(core_axis_name="c", subcore_axis_name="s")

    @functools.partial(
        pl.kernel,
        mesh=mesh,
        out_type=jax.ShapeDtypeStruct((NW, LANES), jnp.float32),
        scratch_types=[
            pltpu.VMEM((CHUNK,), jnp.int32),
            pltpu.VMEM((CHUNK, V), jnp.float32),
            pltpu.VMEM((CHUNK, VC), jnp.float32),
            pltpu.VMEM((LANES,), jnp.float32),
            pltpu.SemaphoreType.DMA,
        ],
    )
    def dot_gather(logits_hbm, labels_hbm, s_hbm, out_hbm,
                   idx_v, x_v, srows_v, acc_v, sem):
        c = lax.axis_index("c")
        s = lax.axis_index("s")
        wid = c * NS + s

        iota = jax.lax.broadcasted_iota(jnp.int32, (LANES,), 0)
        tail_mask = jnp.where(iota >= 8,
                              jnp.ones((LANES,), jnp.float32),
                              jnp.zeros((LANES,), jnp.float32))

        def body(k, acc):
            base = row0 + wid * rows_per_w + k * CHUNK
            pltpu.sync_copy(labels_hbm.at[pl.ds(base, CHUNK)], idx_v)
            stg = pltpu.async_copy(
                logits_hbm.at[pl.ds(base, CHUNK)], x_v, sem)
            gat = pltpu.async_copy(s_hbm.at[idx_v], srows_v, sem)
            stg.wait()
            gat.wait()

            def rbody(r, acc_in):
                a = acc_in
                for j in range(V // LANES):
                    x = x_v[r, pl.ds(j * LANES, LANES)]
                    sr = srows_v[r, pl.ds(j * LANES, LANES)]
                    a = a + x * sr
                xt = x_v[r, pl.ds(V - LANES, LANES)] * tail_mask
                st = srows_v[r, pl.ds(V - LANES, LANES)]
                return a + xt * st

            return lax.fori_loop(0, CHUNK, rbody, acc)

        acc = lax.fori_loop(0, nchunk, body, jnp.zeros((LANES,), jnp.float32))
        acc_v[...] = acc
        pltpu.sync_copy(acc_v, out_hbm.at[wid])

    return dot_gather


def _lse_body(logits_ref, labels_ref, sbf_ref, out_ref, acc_scr):
    i = pl.program_id(0)

    @pl.when(i == 0)
    def _init():
        acc_scr[0] = 0.0

    x = logits_ref[...]
    m = jnp.max(x, axis=1, keepdims=True)
    e = jnp.exp(x - m)
    sm = jnp.sum(e, axis=1)
    lse = jnp.log(sm) + m[:, 0]
    acc_scr[0] += jnp.sum(lse)

    @pl.when(i < TC_BLOCKS)
    def _dot():
        lab = labels_ref[0, 0, :]
        oh = (jax.lax.broadcasted_iota(jnp.int32, (BLK, V), 1)
              == lab[:, None]).astype(jnp.bfloat16)
        sg = jax.lax.dot_general(
            oh, sbf_ref[...], (((1,), (0,)), ((), ())),
            preferred_element_type=jnp.float32)
        acc_scr[0] += -jnp.sum(sg * x)

    @pl.when(i == pl.num_programs(0) - 1)
    def _fin():
        out_ref[0] = acc_scr[0]


def kernel(logits, labels, word_emb_tab):
    logits = logits.astype(jnp.float32)
    labels = labels.astype(jnp.int32)
    batch = logits.shape[0]
    nblk = batch // BLK

    s_tab, s_bf = pl.pallas_call(
        _s_table_body,
        out_shape=(jax.ShapeDtypeStruct((V, VC), jnp.float32),
                   jax.ShapeDtypeStruct((V, V), jnp.bfloat16)),
    )(word_emb_tab.astype(jnp.float32))

    dots = _make_dot_gather(batch)(logits, labels, s_tab)

    labels3 = labels.reshape(nblk, 1, BLK)
    lse_total = pl.pallas_call(
        _lse_body,
        grid=(nblk,),
        in_specs=[
            pl.BlockSpec((BLK, V), lambda i: (i, 0)),
            pl.BlockSpec((1, 1, BLK), lambda i: (i, 0, 0)),
            pl.BlockSpec((V, V), lambda i: (0, 0)),
        ],
        out_specs=pl.BlockSpec(memory_space=pltpu.SMEM),
        out_shape=jax.ShapeDtypeStruct((1,), jnp.float32),
        scratch_shapes=[
            pltpu.SMEM((1,), jnp.float32),
        ],
    )(logits, labels3, s_bf)

    return ((lse_total[0] - jnp.sum(dots)) / batch).astype(jnp.float32)


# speedup vs baseline: 1.3495x; 1.0220x over previous
import functools

import jax
import jax.numpy as jnp
from jax import lax
from jax.experimental import pallas as pl
from jax.experimental.pallas import tpu as pltpu
from jax.experimental.pallas import tpu_sc as plsc

V = 1000
VC = 1024
D = 128
EPS = 0.2
T_INV = 2.0
BLK = 512

NC = 2
NS = 16
NW = NC * NS
CHUNK = 32
TC_BLOCKS = 16
LANES = 16


def _s_table_body(emb_ref, s_ref, sbf_ref):
    emb = emb_ref[...]
    ss = jnp.sum(emb * emb, axis=1, keepdims=True)
    nrm = jnp.maximum(jnp.sqrt(ss), 1e-12)
    emb_n = emb / nrm
    sim = jax.lax.dot_general(
        emb_n, emb_n, (((1,), (1,)), ((), ())),
        preferred_element_type=jnp.float32)
    rows = jax.lax.broadcasted_iota(jnp.int32, (V, V), 0)
    cols = jax.lax.broadcasted_iota(jnp.int32, (V, V), 1)
    diag = rows == cols
    masked = jnp.where(diag, -jnp.inf, sim * T_INV)
    m = jnp.max(masked, axis=1, keepdims=True)
    e = jnp.exp(masked - m)
    denom = jnp.sum(e, axis=1, keepdims=True)
    s = e * (EPS / denom)
    s = jnp.where(diag, 1.0 - EPS, s)
    s_ref[...] = jnp.pad(s, ((0, 0), (0, VC - V)))
    sbf_ref[...] = s.astype(jnp.bfloat16)


def _make_dot_gather(batch):
    row0 = TC_BLOCKS * BLK
    rows_per_w = (batch - row0) // NW
    nchunk = rows_per_w // CHUNK
    mesh = plsc.VectorSubcoreMesh(core_axis_name="c", subcore_axis_name="s")

    @functools.partial(
        pl.kernel,
        mesh=mesh,
        out_type=jax.ShapeDtypeStruct((NW, LANES), jnp.float32),
        scratch_types=[
            pltpu.VMEM((CHUNK,), jnp.int32),
            pltpu.VMEM((CHUNK, V), jnp.float32),
            pltpu.VMEM((CHUNK, VC), jnp.float32),
            pltpu.VMEM((LANES,), jnp.float32),
            pltpu.SemaphoreType.DMA,
        ],
    )
    def dot_gather(logits_hbm, labels_hbm, s_hbm, out_hbm,
                   idx_v, x_v, srows_v, acc_v, sem):
        c = lax.axis_index("c")
        s = lax.axis_index("s")
        wid = c * NS + s

        iota = jax.lax.broadcasted_iota(jnp.int32, (LANES,), 0)
        tail_mask = jnp.where(iota >= 8,
                              jnp.ones((LANES,), jnp.float32),
                              jnp.zeros((LANES,), jnp.float32))

        def body(k, acc):
            base = row0 + wid * rows_per_w + k * CHUNK
            pltpu.sync_copy(labels_hbm.at[pl.ds(base, CHUNK)], idx_v)
            stg = pltpu.async_copy(
                logits_hbm.at[pl.ds(base, CHUNK)], x_v, sem)
            gat = pltpu.async_copy(s_hbm.at[idx_v], srows_v, sem)
            stg.wait()
            gat.wait()

            def rbody(r, acc_in):
                a = acc_in
                for j in range(V // LANES):
                    x = x_v[r, pl.ds(j * LANES, LANES)]
                    sr = srows_v[r, pl.ds(j * LANES, LANES)]
                    a = a + x * sr
                xt = x_v[r, pl.ds(V - LANES, LANES)] * tail_mask
                st = srows_v[r, pl.ds(V - LANES, LANES)]
                return a + xt * st

            return lax.fori_loop(0, CHUNK, rbody, acc)

        acc = lax.fori_loop(0, nchunk, body, jnp.zeros((LANES,), jnp.float32))
        acc_v[...] = acc
        pltpu.sync_copy(acc_v, out_hbm.at[wid])

    return dot_gather


def _lse_body(logits_ref, labels_ref, sbf_ref, out_ref, acc_scr):
    i = pl.program_id(0)

    @pl.when(i == 0)
    def _init():
        acc_scr[0] = 0.0

    x = logits_ref[...]
    m = jnp.max(x, axis=1, keepdims=True)
    e = jnp.exp(x - m)
    sm = jnp.sum(e, axis=1)
    lse = jnp.log(sm) + m[:, 0]
    acc_scr[0] += jnp.sum(lse)

    @pl.when(i < TC_BLOCKS)
    def _dot():
        lab = labels_ref[0, 0, :]
        oh = (jax.lax.broadcasted_iota(jnp.int32, (BLK, V), 1)
              == lab[:, None]).astype(jnp.bfloat16)
        sg = jax.lax.dot_general(
            oh, sbf_ref[...], (((1,), (0,)), ((), ())),
            preferred_element_type=jnp.float32)
        acc_scr[0] += -jnp.sum(sg * x)

    @pl.when(i == pl.num_programs(0) - 1)
    def _fin():
        out_ref[0] = acc_scr[0]


def kernel(logits, labels, word_emb_tab):
    logits = logits.astype(jnp.float32)
    labels = labels.astype(jnp.int32)
    batch = logits.shape[0]
    nblk = batch // BLK

    s_tab, s_bf = pl.pallas_call(
        _s_table_body,
        out_shape=(jax.ShapeDtypeStruct((V, VC), jnp.float32),
                   jax.ShapeDtypeStruct((V, V), jnp.bfloat16)),
    )(word_emb_tab.astype(jnp.float32))

    dots = _make_dot_gather(batch)(logits, labels, s_tab)

    labels3 = labels.reshape(nblk, 1, BLK)
    lse_total = pl.pallas_call(
        _lse_body,
        grid=(nblk,),
        in_specs=[
            pl.BlockSpec((BLK, V), lambda i: (i, 0)),
            pl.BlockSpec((1, 1, BLK), lambda i: (i, 0, 0)),
            pl.BlockSpec((V, V), lambda i: (0, 0)),
        ],
        out_specs=pl.BlockSpec(memory_space=pltpu.SMEM),
        out_shape=jax.ShapeDtypeStruct((1,), jnp.float32),
        scratch_shapes=[
            pltpu.SMEM((1,), jnp.float32),
        ],
    )(logits, labels3, s_bf)

    return ((lse_total[0] - jnp.sum(dots)) / batch).astype(jnp.float32)
